# Initial kernel scaffold; baseline (speedup 1.0000x reference)
import jax
import jax.numpy as jnp
from jax.experimental import pallas as pl


def kernel(atomwise_predictions, atoms, atom_refs):
    raise NotImplementedError("write your pallas kernel here")


# speedup vs baseline: 3.2539x; 3.2539x over previous
import jax
import jax.numpy as jnp
from jax import lax
from jax.experimental import pallas as pl
from jax.experimental.pallas import tpu as pltpu
from jax.experimental.pallas import tpu_sc as plsc

_N = 100000
_NW = 32
_ROWS = 3128
_FLAT = 2 * _ROWS
_VREGS = _FLAT // 16


def _sc_body(ap_hbm, atoms_hbm, table_hbm, out_hbm, atoms_v, ap_v, out_v, table_v):
    c = lax.axis_index("c")
    s = lax.axis_index("s")
    w = s * 2 + c
    base = jnp.minimum(w * _ROWS, _N - _ROWS)
    pltpu.sync_copy(table_hbm, table_v)
    pltpu.sync_copy(atoms_hbm.at[pl.ds(base, _ROWS)], atoms_v)
    pltpu.sync_copy(ap_hbm.at[pl.ds(2 * base, _FLAT)], ap_v)

    lane = lax.iota(jnp.int32, 16)
    col0 = (lane & 1) == 0
    half = lane >> 1

    def step(j, carry):
        row = j * 8 + half
        aidx = plsc.load_gather(atoms_v, [row])
        refv = plsc.load_gather(table_v, [aidx])
        x = ap_v[pl.ds(j * 16, 16)]
        u = jnp.exp(-jnp.abs(x))
        t = u / (u + 2.0)
        t2 = t * t
        poly = 1.0 + t2 * (0.33333334 + t2 * (0.2 + t2 * 0.14285715))
        sp = jnp.maximum(x, 0.0) + 2.0 * t * poly + 1e-8
        out_v[pl.ds(j * 16, 16)] = jnp.where(col0, x + refv, sp)
        return carry

    lax.fori_loop(0, _VREGS, step, 0)
    pltpu.sync_copy(out_v, out_hbm.at[pl.ds(2 * base, _FLAT)])


def kernel(atomwise_predictions, atoms, atom_refs):
    ap_flat = atomwise_predictions.reshape(-1)
    table = jnp.pad(atom_refs.reshape(-1), (0, 28))
    mesh = plsc.VectorSubcoreMesh(core_axis_name="c", subcore_axis_name="s")
    f = pl.kernel(
        _sc_body,
        mesh=mesh,
        compiler_params=pltpu.CompilerParams(needs_layout_passes=False),
        out_type=jax.ShapeDtypeStruct((2 * _N,), jnp.float32),
        scratch_types=[
            pltpu.VMEM((_ROWS,), jnp.int32),
            pltpu.VMEM((_FLAT,), jnp.float32),
            pltpu.VMEM((_FLAT,), jnp.float32),
            pltpu.VMEM((128,), jnp.float32),
        ],
    )
    out = f(ap_flat, atoms, table)
    return out.reshape(_N, 2)


# speedup vs baseline: 3.4729x; 1.0673x over previous
import jax
import jax.numpy as jnp
from jax import lax
from jax.experimental import pallas as pl
from jax.experimental.pallas import tpu as pltpu
from jax.experimental.pallas import tpu_sc as plsc

_N = 100000
_NW = 32
_ROWS = 3136
_FLAT = 2 * _ROWS
_VREGS = _FLAT // 16


def _sc_body(ap_hbm, atoms_hbm, table_hbm, out_hbm, atoms_v, ap_v, out_v, table_v):
    c = lax.axis_index("c")
    s = lax.axis_index("s")
    w = s * 2 + c
    base = jnp.minimum(w * _ROWS, _N - _ROWS)
    pltpu.sync_copy(table_hbm, table_v)
    pltpu.sync_copy(atoms_hbm.at[pl.ds(base, _ROWS)], atoms_v)
    pltpu.sync_copy(ap_hbm.at[pl.ds(2 * base, _FLAT)], ap_v)

    lane = lax.iota(jnp.int32, 16)
    col0 = (lane & 1) == 0
    half = lane >> 1

    @plsc.parallel_loop(0, _VREGS, step=1, unroll=8)
    def _(j):
        row = j * 8 + half
        aidx = plsc.load_gather(atoms_v, [row])
        refv = plsc.load_gather(table_v, [aidx])
        x = ap_v[pl.ds(j * 16, 16)]
        u = jnp.exp(-jnp.abs(x))
        t = u / (u + 2.0)
        t2 = t * t
        poly = 1.0 + t2 * (0.33333334 + t2 * (0.2 + t2 * 0.14285715))
        sp = jnp.maximum(x, 0.0) + 2.0 * t * poly + 1e-8
        out_v[pl.ds(j * 16, 16)] = jnp.where(col0, x + refv, sp)
    pltpu.sync_copy(out_v, out_hbm.at[pl.ds(2 * base, _FLAT)])


def kernel(atomwise_predictions, atoms, atom_refs):
    ap_flat = atomwise_predictions.reshape(-1)
    table = jnp.pad(atom_refs.reshape(-1), (0, 28))
    mesh = plsc.VectorSubcoreMesh(core_axis_name="c", subcore_axis_name="s")
    f = pl.kernel(
        _sc_body,
        mesh=mesh,
        compiler_params=pltpu.CompilerParams(needs_layout_passes=False),
        out_type=jax.ShapeDtypeStruct((2 * _N,), jnp.float32),
        scratch_types=[
            pltpu.VMEM((_ROWS,), jnp.int32),
            pltpu.VMEM((_FLAT,), jnp.float32),
            pltpu.VMEM((_FLAT,), jnp.float32),
            pltpu.VMEM((128,), jnp.float32),
        ],
    )
    out = f(ap_flat, atoms, table)
    return out.reshape(_N, 2)
